# Initial kernel scaffold; baseline (speedup 1.0000x reference)
import jax
import jax.numpy as jnp
from jax.experimental import pallas as pl


def kernel(xyz, points, w0, b0, g0, be0, w1, b1, g1, be1, w2, b2, g2, be2):
    raise NotImplementedError("write your pallas kernel here")


# speedup vs baseline: 5.0050x; 5.0050x over previous
import functools

import jax
import jax.numpy as jnp
from jax import lax
from jax.experimental import pallas as pl
from jax.experimental.pallas import tpu as pltpu
from jax.experimental.pallas import tpu_sc as plsc

NPOINT = 512
K = 32
N = 4096
B = 8
C_IN = 32
EPS = 1e-5
BSK = B * NPOINT * K


def _fps_knn_kernel(xyz_ref, xyzr_ref, xt_ref, pts_ref, w0x_ref, w0p_ref,
                    newxyz_ref, gidx_ref, z_ref, qw_ref, dmat_ref):
    b = pl.program_id(0)

    x3 = xyzr_ref[0]
    x0 = x3[0]
    x1 = x3[1]
    x2 = x3[2]
    row_i = lax.broadcasted_iota(jnp.int32, (8, 512), 0)
    col_i = lax.broadcasted_iota(jnp.int32, (8, 512), 1)
    lin = row_i * 512 + col_i

    def fps_body(i, carry):
        dist, far = carry
        rowv = xt_ref[pl.ds(far, 1), :]
        newxyz_ref[pl.ds(i, 1), :] = rowv
        cx = jnp.sum(rowv[:, 0:1])
        cy = jnp.sum(rowv[:, 1:2])
        cz = jnp.sum(rowv[:, 2:3])
        d = (x0 - cx) ** 2 + (x1 - cy) ** 2 + (x2 - cz) ** 2
        dist = jnp.minimum(dist, d)
        m = jnp.max(dist)
        far2 = jnp.min(jnp.where(dist == m, lin, N)).astype(jnp.int32)
        return dist, far2

    dist0 = jnp.full((8, 512), 1e10, jnp.float32)
    lax.fori_loop(0, NPOINT, fps_body, (dist0, jnp.int32(0)))

    q = newxyz_ref[...]
    x = xyz_ref[0]
    dots = lax.dot_general(q, x, (((1,), (0,)), ((), ())),
                           preferred_element_type=jnp.float32)
    qq = jnp.sum(q * q, axis=1, keepdims=True)
    xx = jnp.sum(x * x, axis=0, keepdims=True)
    dmat_ref[...] = (qq - 2.0 * dots) + xx

    lane_full = lax.broadcasted_iota(jnp.int32, (64, N), 1)
    col32 = lax.broadcasted_iota(jnp.int32, (64, K), 1)
    inf = jnp.float32(3.0e38)

    for t in range(NPOINT // 64):
        rows = pl.ds(t * 64, 64)

        def knn_body(j, acc):
            dt = dmat_ref[rows, :]
            m = jnp.min(dt, axis=1, keepdims=True)
            idx = jnp.min(jnp.where(dt == m, lane_full, N), axis=1,
                          keepdims=True).astype(jnp.int32)
            dmat_ref[rows, :] = jnp.where(lane_full == idx, inf, dt)
            return jnp.where(col32 == j, idx, acc)

        acc0 = jnp.zeros((64, K), jnp.int32)
        acc = lax.fori_loop(0, K, knn_body, acc0)
        gidx_ref[rows, :] = acc + b * N

    w0x = w0x_ref[...]
    z = (lax.dot_general(xt_ref[...], w0x, (((1,), (0,)), ((), ())),
                         preferred_element_type=jnp.float32)
         + lax.dot_general(pts_ref[0], w0p_ref[...], (((1,), (0,)), ((), ())),
                           preferred_element_type=jnp.float32))
    z_ref[...] = jnp.concatenate(
        [z, jnp.zeros((N, 96), jnp.float32)], axis=1)
    qw_ref[...] = lax.dot_general(q, w0x, (((1,), (0,)), ((), ())),
                                  preferred_element_type=jnp.float32)


def _run_fps_knn(xyz, xyz_r, xyz_t, points_t, w0x, w0p):
    return pl.pallas_call(
        _fps_knn_kernel,
        grid=(B,),
        in_specs=[
            pl.BlockSpec((1, 3, N), lambda b: (b, 0, 0)),
            pl.BlockSpec((1, 3, 8, 512), lambda b: (b, 0, 0, 0)),
            pl.BlockSpec((N, 3), lambda b: (b, 0)),
            pl.BlockSpec((1, N, C_IN), lambda b: (b, 0, 0)),
            pl.BlockSpec((3, 32), lambda b: (0, 0)),
            pl.BlockSpec((C_IN, 32), lambda b: (0, 0)),
        ],
        out_specs=[
            pl.BlockSpec((NPOINT, 3), lambda b: (b, 0)),
            pl.BlockSpec((NPOINT, K), lambda b: (b, 0)),
            pl.BlockSpec((N, 128), lambda b: (b, 0)),
            pl.BlockSpec((NPOINT, 32), lambda b: (b, 0)),
        ],
        out_shape=[
            jax.ShapeDtypeStruct((B * NPOINT, 3), jnp.float32),
            jax.ShapeDtypeStruct((B * NPOINT, K), jnp.int32),
            jax.ShapeDtypeStruct((B * N, 128), jnp.float32),
            jax.ShapeDtypeStruct((B * NPOINT, 32), jnp.float32),
        ],
        scratch_shapes=[pltpu.VMEM((NPOINT, N), jnp.float32)],
    )(xyz, xyz_r, xyz_t, points_t, w0x, w0p)


_SC_NW = 32
_SC_BPW = (B * NPOINT * K) // _SC_NW
_SC_CHUNK = 512


def _sc_gather_kernel(z_hbm, idx_hbm, out_hbm, idx_v, rows_v, sem):
    wid = lax.axis_index("s") * 2 + lax.axis_index("c")
    base = wid * _SC_BPW
    pltpu.sync_copy(idx_hbm.at[pl.ds(base, _SC_BPW)], idx_v)
    for c in range(_SC_BPW // _SC_CHUNK):
        pltpu.async_copy(
            z_hbm.at[idx_v.at[pl.ds(c * _SC_CHUNK, _SC_CHUNK)]],
            rows_v, sem).wait()
        pltpu.sync_copy(rows_v,
                        out_hbm.at[pl.ds(base + c * _SC_CHUNK, _SC_CHUNK)])


def _run_sc_gather(z_flat, idx_flat):
    mesh = plsc.VectorSubcoreMesh(core_axis_name="c", subcore_axis_name="s")
    k = functools.partial(
        pl.kernel, mesh=mesh,
        out_type=jax.ShapeDtypeStruct((B * NPOINT * K, 128), jnp.float32),
        scratch_types=[
            pltpu.VMEM((_SC_BPW,), jnp.int32),
            pltpu.VMEM((_SC_CHUNK, 128), jnp.float32),
            pltpu.SemaphoreType.DMA,
        ],
    )(_sc_gather_kernel)
    return k(z_flat, idx_flat)


def _stats_of(y):
    if y.ndim == 3:
        s = jnp.sum(jnp.sum(y, axis=1), axis=0, keepdims=True)
        s2 = jnp.sum(jnp.sum(y * y, axis=1), axis=0, keepdims=True)
    else:
        s = jnp.sum(y, axis=0, keepdims=True)
        s2 = jnp.sum(y * y, axis=0, keepdims=True)
    return s, s2


def _layer0_kernel(g_ref, qw_ref, b0_ref, y_ref, st_ref):
    b = pl.program_id(0)
    g = g_ref[0][:, :, 0:32]
    qw = qw_ref[0]
    y = g - qw[:, None, :] + b0_ref[...][None, :, :]
    y_ref[0] = y
    s, s2 = _stats_of(y)

    @pl.when(b == 0)
    def _():
        st_ref[...] = jnp.zeros_like(st_ref)

    st_ref[0:1, :] += s
    st_ref[1:2, :] += s2


def _run_layer0(g4, qw3, b0):
    return pl.pallas_call(
        _layer0_kernel,
        grid=(B,),
        in_specs=[
            pl.BlockSpec((1, NPOINT, K, 128), lambda b: (b, 0, 0, 0)),
            pl.BlockSpec((1, NPOINT, 32), lambda b: (b, 0, 0)),
            pl.BlockSpec((1, 32), lambda b: (0, 0)),
        ],
        out_specs=[
            pl.BlockSpec((1, NPOINT, K, 32), lambda b: (b, 0, 0, 0)),
            pl.BlockSpec((2, 32), lambda b: (0, 0)),
        ],
        out_shape=[
            jax.ShapeDtypeStruct((B, NPOINT, K, 32), jnp.float32),
            jax.ShapeDtypeStruct((2, 32), jnp.float32),
        ],
    )(g4, qw3, b0)


def _bn_scale_shift(st_ref, g_ref, be_ref):
    mean = st_ref[0:1, :] / BSK
    var = st_ref[1:2, :] / BSK - mean * mean
    scale = g_ref[...] / jnp.sqrt(var + EPS)
    shift = be_ref[...] - mean * scale
    return scale, shift


def _mid_layer_kernel(y_ref, st_ref, g_ref, be_ref, wt_ref, bias_ref,
                      yo_ref, sto_ref):
    b = pl.program_id(0)
    scale, shift = _bn_scale_shift(st_ref, g_ref, be_ref)
    x = jnp.maximum(y_ref[0] * scale + shift, 0.0)
    y = lax.dot_general(x, wt_ref[...], (((1,), (0,)), ((), ())),
                        preferred_element_type=jnp.float32)
    y = y + bias_ref[...]
    yo_ref[0] = y
    s, s2 = _stats_of(y)

    @pl.when(b == 0)
    def _():
        sto_ref[...] = jnp.zeros_like(sto_ref)

    sto_ref[0:1, :] += s
    sto_ref[1:2, :] += s2


def _run_mid_layer(y2d, st, g, be, wt, bias, c_out):
    c_in = y2d.shape[-1]
    return pl.pallas_call(
        _mid_layer_kernel,
        grid=(B,),
        in_specs=[
            pl.BlockSpec((1, NPOINT * K, c_in), lambda b: (b, 0, 0)),
            pl.BlockSpec((2, c_in), lambda b: (0, 0)),
            pl.BlockSpec((1, c_in), lambda b: (0, 0)),
            pl.BlockSpec((1, c_in), lambda b: (0, 0)),
            pl.BlockSpec((c_in, c_out), lambda b: (0, 0)),
            pl.BlockSpec((1, c_out), lambda b: (0, 0)),
        ],
        out_specs=[
            pl.BlockSpec((1, NPOINT * K, c_out), lambda b: (b, 0, 0)),
            pl.BlockSpec((2, c_out), lambda b: (0, 0)),
        ],
        out_shape=[
            jax.ShapeDtypeStruct((B, NPOINT * K, c_out), jnp.float32),
            jax.ShapeDtypeStruct((2, c_out), jnp.float32),
        ],
    )(y2d, st, g, be, wt, bias)


def _final_kernel(y_ref, st_ref, g_ref, be_ref, out_ref):
    scale, shift = _bn_scale_shift(st_ref, g_ref, be_ref)
    x = jnp.maximum(y_ref[0] * scale[None] + shift[None], 0.0)
    out_ref[0] = jnp.max(x, axis=1)


def _run_final(y3d, st, g, be):
    return pl.pallas_call(
        _final_kernel,
        grid=(B,),
        in_specs=[
            pl.BlockSpec((1, NPOINT, K, 64), lambda b: (b, 0, 0, 0)),
            pl.BlockSpec((2, 64), lambda b: (0, 0)),
            pl.BlockSpec((1, 64), lambda b: (0, 0)),
            pl.BlockSpec((1, 64), lambda b: (0, 0)),
        ],
        out_specs=pl.BlockSpec((1, NPOINT, 64), lambda b: (b, 0, 0)),
        out_shape=jax.ShapeDtypeStruct((B, NPOINT, 64), jnp.float32),
    )(y3d, st, g, be)


def kernel(xyz, points, w0, b0, g0, be0, w1, b1, g1, be1, w2, b2, g2, be2):
    xyz_r = xyz.reshape(B, 3, 8, 512)
    xyz_t = jnp.transpose(xyz, (0, 2, 1)).reshape(B * N, 3)
    points_t = jnp.transpose(points, (0, 2, 1))
    w0t = jnp.transpose(w0)
    w0x, w0p = w0t[:3], w0t[3:]

    new_xyz, gidx, z_flat, qw = _run_fps_knn(xyz, xyz_r, xyz_t, points_t,
                                             w0x, w0p)
    new_xyz = new_xyz.reshape(B, NPOINT, 3)
    qw3 = qw.reshape(B, NPOINT, 32)

    gathered = _run_sc_gather(z_flat, gidx.reshape(-1))
    g4 = gathered.reshape(B, NPOINT, K, 128)

    y0, st0 = _run_layer0(g4, qw3, b0.reshape(1, 32))
    y1, st1 = _run_mid_layer(y0.reshape(B, NPOINT * K, 32), st0,
                             g0.reshape(1, 32), be0.reshape(1, 32),
                             jnp.transpose(w1), b1.reshape(1, 32), 32)
    y2, st2 = _run_mid_layer(y1, st1, g1.reshape(1, 32), be1.reshape(1, 32),
                             jnp.transpose(w2), b2.reshape(1, 64), 64)
    out = _run_final(y2.reshape(B, NPOINT, K, 64), st2,
                     g2.reshape(1, 64), be2.reshape(1, 64))

    new_xyz_out = jnp.transpose(new_xyz, (0, 2, 1))
    agg = jnp.transpose(out, (0, 2, 1))
    return (new_xyz_out, agg)


# speedup vs baseline: 13.1110x; 2.6196x over previous
import functools

import jax
import jax.numpy as jnp
from jax import lax
from jax.experimental import pallas as pl
from jax.experimental.pallas import tpu as pltpu
from jax.experimental.pallas import tpu_sc as plsc

NPOINT = 512
K = 32
N = 4096
B = 8
C_IN = 32
EPS = 1e-5
BSK = B * NPOINT * K


def _fps_kernel(xyzc_ref, newxyz_ref):
    xc = xyzc_ref[...]
    x0 = xc[0]
    x1 = xc[1]
    x2 = xc[2]
    lane = lax.broadcasted_iota(jnp.int32, (B, N), 1)

    def fps_body(i, carry):
        dist, far = carry
        oh = lane == far
        cx = jnp.sum(jnp.where(oh, x0, 0.0), axis=1, keepdims=True)
        cy = jnp.sum(jnp.where(oh, x1, 0.0), axis=1, keepdims=True)
        cz = jnp.sum(jnp.where(oh, x2, 0.0), axis=1, keepdims=True)
        newxyz_ref[pl.ds(i, 1)] = jnp.concatenate([cx, cy, cz], axis=1)[None]
        d = (x0 - cx) ** 2 + (x1 - cy) ** 2 + (x2 - cz) ** 2
        dist = jnp.minimum(dist, d)
        m = jnp.max(dist, axis=1, keepdims=True)
        far2 = jnp.min(jnp.where(dist == m, lane, N), axis=1,
                       keepdims=True).astype(jnp.int32)
        return dist, far2

    dist0 = jnp.full((B, N), 1e10, jnp.float32)
    far0 = jnp.zeros((B, 1), jnp.int32)
    lax.fori_loop(0, NPOINT, fps_body, (dist0, far0))


def _run_fps(xyz_c):
    return pl.pallas_call(
        _fps_kernel,
        out_shape=jax.ShapeDtypeStruct((NPOINT, B, 3), jnp.float32),
    )(xyz_c)


def _knn_kernel(xyz_ref, nxyz_ref, xt_ref, pts_ref, w0x_ref, w0p_ref,
                gidx_ref, z_ref, qw_ref, dmat_ref):
    b = pl.program_id(0)

    q = nxyz_ref[...]
    x = xyz_ref[0]
    dots = lax.dot_general(q, x, (((1,), (0,)), ((), ())),
                           preferred_element_type=jnp.float32)
    qq = jnp.sum(q * q, axis=1, keepdims=True)
    xx = jnp.sum(x * x, axis=0, keepdims=True)
    dmat_ref[...] = (qq - 2.0 * dots) + xx

    lane_full = lax.broadcasted_iota(jnp.int32, (64, N), 1)
    col32 = lax.broadcasted_iota(jnp.int32, (64, K), 1)
    inf = jnp.float32(3.0e38)
    NT = NPOINT // 64

    def knn_body(j, accs):
        new_accs = []
        for t in range(NT):
            rows = pl.ds(t * 64, 64)
            dt = dmat_ref[rows, :]
            m = jnp.min(dt, axis=1, keepdims=True)
            idx = jnp.min(jnp.where(dt == m, lane_full, N), axis=1,
                          keepdims=True).astype(jnp.int32)
            dmat_ref[rows, :] = jnp.where(lane_full == idx, inf, dt)
            new_accs.append(jnp.where(col32 == j, idx, accs[t]))
        return tuple(new_accs)

    acc0 = tuple(jnp.zeros((64, K), jnp.int32) for _ in range(NT))
    accs = lax.fori_loop(0, K, knn_body, acc0)
    for t in range(NT):
        gidx_ref[pl.ds(t * 64, 64), :] = accs[t] + b * N

    w0x = w0x_ref[...]
    z = (lax.dot_general(xt_ref[...], w0x, (((1,), (0,)), ((), ())),
                         preferred_element_type=jnp.float32)
         + lax.dot_general(pts_ref[0], w0p_ref[...], (((1,), (0,)), ((), ())),
                           preferred_element_type=jnp.float32))
    z_ref[...] = jnp.concatenate(
        [z, jnp.zeros((N, 96), jnp.float32)], axis=1)
    qw_ref[...] = lax.dot_general(q, w0x, (((1,), (0,)), ((), ())),
                                  preferred_element_type=jnp.float32)


def _run_knn(xyz, new_xyz_flat, xyz_t, points_t, w0x, w0p):
    return pl.pallas_call(
        _knn_kernel,
        grid=(B,),
        in_specs=[
            pl.BlockSpec((1, 3, N), lambda b: (b, 0, 0)),
            pl.BlockSpec((NPOINT, 3), lambda b: (b, 0)),
            pl.BlockSpec((N, 3), lambda b: (b, 0)),
            pl.BlockSpec((1, N, C_IN), lambda b: (b, 0, 0)),
            pl.BlockSpec((3, 32), lambda b: (0, 0)),
            pl.BlockSpec((C_IN, 32), lambda b: (0, 0)),
        ],
        out_specs=[
            pl.BlockSpec((NPOINT, K), lambda b: (b, 0)),
            pl.BlockSpec((N, 128), lambda b: (b, 0)),
            pl.BlockSpec((NPOINT, 32), lambda b: (b, 0)),
        ],
        out_shape=[
            jax.ShapeDtypeStruct((B * NPOINT, K), jnp.int32),
            jax.ShapeDtypeStruct((B * N, 128), jnp.float32),
            jax.ShapeDtypeStruct((B * NPOINT, 32), jnp.float32),
        ],
        scratch_shapes=[pltpu.VMEM((NPOINT, N), jnp.float32)],
    )(xyz, new_xyz_flat, xyz_t, points_t, w0x, w0p)


_SC_NW = 32
_SC_BPW = (B * NPOINT * K) // _SC_NW
_SC_CHUNK = 512


def _sc_gather_kernel(z_hbm, idx_hbm, out_hbm, idx_v, rows_v, sem):
    wid = lax.axis_index("s") * 2 + lax.axis_index("c")
    base = wid * _SC_BPW
    pltpu.sync_copy(idx_hbm.at[pl.ds(base, _SC_BPW)], idx_v)
    for c in range(_SC_BPW // _SC_CHUNK):
        pltpu.async_copy(
            z_hbm.at[idx_v.at[pl.ds(c * _SC_CHUNK, _SC_CHUNK)]],
            rows_v, sem).wait()
        pltpu.sync_copy(rows_v,
                        out_hbm.at[pl.ds(base + c * _SC_CHUNK, _SC_CHUNK)])


def _run_sc_gather(z_flat, idx_flat):
    mesh = plsc.VectorSubcoreMesh(core_axis_name="c", subcore_axis_name="s")
    k = functools.partial(
        pl.kernel, mesh=mesh,
        out_type=jax.ShapeDtypeStruct((B * NPOINT * K, 128), jnp.float32),
        scratch_types=[
            pltpu.VMEM((_SC_BPW,), jnp.int32),
            pltpu.VMEM((_SC_CHUNK, 128), jnp.float32),
            pltpu.SemaphoreType.DMA,
        ],
    )(_sc_gather_kernel)
    return k(z_flat, idx_flat)


def _stats_of(y):
    if y.ndim == 3:
        s = jnp.sum(jnp.sum(y, axis=1), axis=0, keepdims=True)
        s2 = jnp.sum(jnp.sum(y * y, axis=1), axis=0, keepdims=True)
    else:
        s = jnp.sum(y, axis=0, keepdims=True)
        s2 = jnp.sum(y * y, axis=0, keepdims=True)
    return s, s2


def _layer0_kernel(g_ref, qw_ref, b0_ref, y_ref, st_ref):
    b = pl.program_id(0)
    g = g_ref[0][:, :, 0:32]
    qw = qw_ref[0]
    y = g - qw[:, None, :] + b0_ref[...][None, :, :]
    y_ref[0] = y
    s, s2 = _stats_of(y)

    @pl.when(b == 0)
    def _():
        st_ref[...] = jnp.zeros_like(st_ref)

    st_ref[0:1, :] += s
    st_ref[1:2, :] += s2


def _run_layer0(g4, qw3, b0):
    return pl.pallas_call(
        _layer0_kernel,
        grid=(B,),
        in_specs=[
            pl.BlockSpec((1, NPOINT, K, 128), lambda b: (b, 0, 0, 0)),
            pl.BlockSpec((1, NPOINT, 32), lambda b: (b, 0, 0)),
            pl.BlockSpec((1, 32), lambda b: (0, 0)),
        ],
        out_specs=[
            pl.BlockSpec((1, NPOINT, K, 32), lambda b: (b, 0, 0, 0)),
            pl.BlockSpec((2, 32), lambda b: (0, 0)),
        ],
        out_shape=[
            jax.ShapeDtypeStruct((B, NPOINT, K, 32), jnp.float32),
            jax.ShapeDtypeStruct((2, 32), jnp.float32),
        ],
    )(g4, qw3, b0)


def _bn_scale_shift(st_ref, g_ref, be_ref):
    mean = st_ref[0:1, :] / BSK
    var = st_ref[1:2, :] / BSK - mean * mean
    scale = g_ref[...] / jnp.sqrt(var + EPS)
    shift = be_ref[...] - mean * scale
    return scale, shift


def _mid_layer_kernel(y_ref, st_ref, g_ref, be_ref, wt_ref, bias_ref,
                      yo_ref, sto_ref):
    b = pl.program_id(0)
    scale, shift = _bn_scale_shift(st_ref, g_ref, be_ref)
    x = jnp.maximum(y_ref[0] * scale + shift, 0.0)
    y = lax.dot_general(x, wt_ref[...], (((1,), (0,)), ((), ())),
                        preferred_element_type=jnp.float32)
    y = y + bias_ref[...]
    yo_ref[0] = y
    s, s2 = _stats_of(y)

    @pl.when(b == 0)
    def _():
        sto_ref[...] = jnp.zeros_like(sto_ref)

    sto_ref[0:1, :] += s
    sto_ref[1:2, :] += s2


def _run_mid_layer(y2d, st, g, be, wt, bias, c_out):
    c_in = y2d.shape[-1]
    return pl.pallas_call(
        _mid_layer_kernel,
        grid=(B,),
        in_specs=[
            pl.BlockSpec((1, NPOINT * K, c_in), lambda b: (b, 0, 0)),
            pl.BlockSpec((2, c_in), lambda b: (0, 0)),
            pl.BlockSpec((1, c_in), lambda b: (0, 0)),
            pl.BlockSpec((1, c_in), lambda b: (0, 0)),
            pl.BlockSpec((c_in, c_out), lambda b: (0, 0)),
            pl.BlockSpec((1, c_out), lambda b: (0, 0)),
        ],
        out_specs=[
            pl.BlockSpec((1, NPOINT * K, c_out), lambda b: (b, 0, 0)),
            pl.BlockSpec((2, c_out), lambda b: (0, 0)),
        ],
        out_shape=[
            jax.ShapeDtypeStruct((B, NPOINT * K, c_out), jnp.float32),
            jax.ShapeDtypeStruct((2, c_out), jnp.float32),
        ],
    )(y2d, st, g, be, wt, bias)


def _final_kernel(y_ref, st_ref, g_ref, be_ref, out_ref):
    scale, shift = _bn_scale_shift(st_ref, g_ref, be_ref)
    x = jnp.maximum(y_ref[0] * scale[None] + shift[None], 0.0)
    out_ref[0] = jnp.max(x, axis=1)


def _run_final(y3d, st, g, be):
    return pl.pallas_call(
        _final_kernel,
        grid=(B,),
        in_specs=[
            pl.BlockSpec((1, NPOINT, K, 64), lambda b: (b, 0, 0, 0)),
            pl.BlockSpec((2, 64), lambda b: (0, 0)),
            pl.BlockSpec((1, 64), lambda b: (0, 0)),
            pl.BlockSpec((1, 64), lambda b: (0, 0)),
        ],
        out_specs=pl.BlockSpec((1, NPOINT, 64), lambda b: (b, 0, 0)),
        out_shape=jax.ShapeDtypeStruct((B, NPOINT, 64), jnp.float32),
    )(y3d, st, g, be)


def kernel(xyz, points, w0, b0, g0, be0, w1, b1, g1, be1, w2, b2, g2, be2):
    xyz_c = jnp.transpose(xyz, (1, 0, 2))
    xyz_t = jnp.transpose(xyz, (0, 2, 1)).reshape(B * N, 3)
    points_t = jnp.transpose(points, (0, 2, 1))
    w0t = jnp.transpose(w0)
    w0x, w0p = w0t[:3], w0t[3:]

    new_xyz = jnp.transpose(_run_fps(xyz_c), (1, 0, 2))

    gidx, z_flat, qw = _run_knn(xyz, new_xyz.reshape(B * NPOINT, 3),
                                xyz_t, points_t, w0x, w0p)
    qw3 = qw.reshape(B, NPOINT, 32)

    gathered = _run_sc_gather(z_flat, gidx.reshape(-1))
    g4 = gathered.reshape(B, NPOINT, K, 128)

    y0, st0 = _run_layer0(g4, qw3, b0.reshape(1, 32))
    y1, st1 = _run_mid_layer(y0.reshape(B, NPOINT * K, 32), st0,
                             g0.reshape(1, 32), be0.reshape(1, 32),
                             jnp.transpose(w1), b1.reshape(1, 32), 32)
    y2, st2 = _run_mid_layer(y1, st1, g1.reshape(1, 32), be1.reshape(1, 32),
                             jnp.transpose(w2), b2.reshape(1, 64), 64)
    out = _run_final(y2.reshape(B, NPOINT, K, 64), st2,
                     g2.reshape(1, 64), be2.reshape(1, 64))

    new_xyz_out = jnp.transpose(new_xyz, (0, 2, 1))
    agg = jnp.transpose(out, (0, 2, 1))
    return (new_xyz_out, agg)


# speedup vs baseline: 13.3247x; 1.0163x over previous
import functools

import jax
import jax.numpy as jnp
from jax import lax
from jax.experimental import pallas as pl
from jax.experimental.pallas import tpu as pltpu
from jax.experimental.pallas import tpu_sc as plsc

NPOINT = 512
K = 32
N = 4096
B = 8
C_IN = 32
EPS = 1e-5
BSK = B * NPOINT * K


def _fps_kernel(xyzc_ref, newxyz_ref):
    xc = xyzc_ref[...]
    x0 = xc[0]
    x1 = xc[1]
    x2 = xc[2]
    lane = lax.broadcasted_iota(jnp.int32, (B, N), 1)

    def fps_body(i, carry):
        dist, far = carry
        oh = lane == far
        cx = jnp.sum(jnp.where(oh, x0, 0.0), axis=1, keepdims=True)
        cy = jnp.sum(jnp.where(oh, x1, 0.0), axis=1, keepdims=True)
        cz = jnp.sum(jnp.where(oh, x2, 0.0), axis=1, keepdims=True)
        newxyz_ref[pl.ds(i, 1)] = jnp.concatenate([cx, cy, cz], axis=1)[None]
        d = (x0 - cx) ** 2 + (x1 - cy) ** 2 + (x2 - cz) ** 2
        dist = jnp.minimum(dist, d)
        m = jnp.max(dist, axis=1, keepdims=True)
        far2 = jnp.min(jnp.where(dist == m, lane, N), axis=1,
                       keepdims=True).astype(jnp.int32)
        return dist, far2

    dist0 = jnp.full((B, N), 1e10, jnp.float32)
    far0 = jnp.zeros((B, 1), jnp.int32)
    lax.fori_loop(0, NPOINT, fps_body, (dist0, far0))


def _run_fps(xyz_c):
    return pl.pallas_call(
        _fps_kernel,
        out_shape=jax.ShapeDtypeStruct((NPOINT, B, 3), jnp.float32),
    )(xyz_c)


def _knn_kernel(xyz_ref, nxyz_ref, pts_ref, w0x_ref, w0p_ref,
                gidx_ref, z_ref, qw_ref, dmat_ref):
    b = pl.program_id(0)

    q = nxyz_ref[...]
    x = xyz_ref[0]
    dots = lax.dot_general(q, x, (((1,), (0,)), ((), ())),
                           preferred_element_type=jnp.float32)
    qq = jnp.sum(q * q, axis=1, keepdims=True)
    xx = jnp.sum(x * x, axis=0, keepdims=True)
    dmat_ref[...] = (qq - 2.0 * dots) + xx

    lane_full = lax.broadcasted_iota(jnp.int32, (64, N), 1)
    col32 = lax.broadcasted_iota(jnp.int32, (64, K), 1)
    inf = jnp.float32(3.0e38)
    NT = NPOINT // 64

    def knn_body(j, accs):
        new_accs = []
        for t in range(NT):
            rows = pl.ds(t * 64, 64)
            dt = dmat_ref[rows, :]
            m = jnp.min(dt, axis=1, keepdims=True)
            idx = jnp.min(jnp.where(dt == m, lane_full, N), axis=1,
                          keepdims=True).astype(jnp.int32)
            dmat_ref[rows, :] = jnp.where(lane_full == idx, inf, dt)
            new_accs.append(jnp.where(col32 == j, idx, accs[t]))
        return tuple(new_accs)

    acc0 = tuple(jnp.zeros((64, K), jnp.int32) for _ in range(NT))
    accs = lax.fori_loop(0, K, knn_body, acc0)
    for t in range(NT):
        gidx_ref[pl.ds(t * 64, 64), :] = accs[t] + b * N

    w0x = w0x_ref[...]
    z = (lax.dot_general(x, w0x, (((0,), (0,)), ((), ())),
                         preferred_element_type=jnp.float32)
         + lax.dot_general(pts_ref[0], w0p_ref[...], (((0,), (0,)), ((), ())),
                           preferred_element_type=jnp.float32))
    z_ref[...] = jnp.concatenate(
        [z, jnp.zeros((N, 96), jnp.float32)], axis=1)
    qw_ref[...] = lax.dot_general(q, w0x, (((1,), (0,)), ((), ())),
                                  preferred_element_type=jnp.float32)


def _run_knn(xyz, new_xyz_flat, points, w0x, w0p):
    return pl.pallas_call(
        _knn_kernel,
        grid=(B,),
        in_specs=[
            pl.BlockSpec((1, 3, N), lambda b: (b, 0, 0)),
            pl.BlockSpec((NPOINT, 3), lambda b: (b, 0)),
            pl.BlockSpec((1, C_IN, N), lambda b: (b, 0, 0)),
            pl.BlockSpec((3, 32), lambda b: (0, 0)),
            pl.BlockSpec((C_IN, 32), lambda b: (0, 0)),
        ],
        out_specs=[
            pl.BlockSpec((NPOINT, K), lambda b: (b, 0)),
            pl.BlockSpec((N, 128), lambda b: (b, 0)),
            pl.BlockSpec((NPOINT, 32), lambda b: (b, 0)),
        ],
        out_shape=[
            jax.ShapeDtypeStruct((B * NPOINT, K), jnp.int32),
            jax.ShapeDtypeStruct((B * N, 128), jnp.float32),
            jax.ShapeDtypeStruct((B * NPOINT, 32), jnp.float32),
        ],
        scratch_shapes=[pltpu.VMEM((NPOINT, N), jnp.float32)],
    )(xyz, new_xyz_flat, points, w0x, w0p)


_SC_NW = 32
_SC_BPW = (B * NPOINT * K) // _SC_NW
_SC_CHUNK = 512


def _sc_gather_kernel(z_hbm, idx_hbm, out_hbm, idx_v, rows_v, sem):
    wid = lax.axis_index("s") * 2 + lax.axis_index("c")
    base = wid * _SC_BPW
    pltpu.sync_copy(idx_hbm.at[pl.ds(base, _SC_BPW)], idx_v)
    for c in range(_SC_BPW // _SC_CHUNK):
        pltpu.async_copy(
            z_hbm.at[idx_v.at[pl.ds(c * _SC_CHUNK, _SC_CHUNK)]],
            rows_v, sem).wait()
        pltpu.sync_copy(rows_v,
                        out_hbm.at[pl.ds(base + c * _SC_CHUNK, _SC_CHUNK)])


def _run_sc_gather(z_flat, idx_flat):
    mesh = plsc.VectorSubcoreMesh(core_axis_name="c", subcore_axis_name="s")
    k = functools.partial(
        pl.kernel, mesh=mesh,
        out_type=jax.ShapeDtypeStruct((B * NPOINT * K, 128), jnp.float32),
        scratch_types=[
            pltpu.VMEM((_SC_BPW,), jnp.int32),
            pltpu.VMEM((_SC_CHUNK, 128), jnp.float32),
            pltpu.SemaphoreType.DMA,
        ],
    )(_sc_gather_kernel)
    return k(z_flat, idx_flat)


def _stats_of(y):
    if y.ndim == 3:
        s = jnp.sum(jnp.sum(y, axis=1), axis=0, keepdims=True)
        s2 = jnp.sum(jnp.sum(y * y, axis=1), axis=0, keepdims=True)
    else:
        s = jnp.sum(y, axis=0, keepdims=True)
        s2 = jnp.sum(y * y, axis=0, keepdims=True)
    return s, s2


def _layer0_kernel(g_ref, qw_ref, b0_ref, y_ref, st_ref):
    b = pl.program_id(0)
    g = g_ref[0][:, :, 0:32]
    qw = qw_ref[0]
    y = g - qw[:, None, :] + b0_ref[...][None, :, :]
    y_ref[0] = y
    s, s2 = _stats_of(y)

    @pl.when(b == 0)
    def _():
        st_ref[...] = jnp.zeros_like(st_ref)

    st_ref[0:1, :] += s
    st_ref[1:2, :] += s2


def _run_layer0(g4, qw3, b0):
    return pl.pallas_call(
        _layer0_kernel,
        grid=(B,),
        in_specs=[
            pl.BlockSpec((1, NPOINT, K, 128), lambda b: (b, 0, 0, 0)),
            pl.BlockSpec((1, NPOINT, 32), lambda b: (b, 0, 0)),
            pl.BlockSpec((1, 32), lambda b: (0, 0)),
        ],
        out_specs=[
            pl.BlockSpec((1, NPOINT, K, 32), lambda b: (b, 0, 0, 0)),
            pl.BlockSpec((2, 32), lambda b: (0, 0)),
        ],
        out_shape=[
            jax.ShapeDtypeStruct((B, NPOINT, K, 32), jnp.float32),
            jax.ShapeDtypeStruct((2, 32), jnp.float32),
        ],
    )(g4, qw3, b0)


def _bn_scale_shift(st_ref, g_ref, be_ref):
    mean = st_ref[0:1, :] / BSK
    var = st_ref[1:2, :] / BSK - mean * mean
    scale = g_ref[...] / jnp.sqrt(var + EPS)
    shift = be_ref[...] - mean * scale
    return scale, shift


def _mid_layer_kernel(y_ref, st_ref, g_ref, be_ref, wt_ref, bias_ref,
                      yo_ref, sto_ref):
    b = pl.program_id(0)
    scale, shift = _bn_scale_shift(st_ref, g_ref, be_ref)
    x = jnp.maximum(y_ref[0] * scale + shift, 0.0)
    y = lax.dot_general(x, wt_ref[...], (((1,), (0,)), ((), ())),
                        preferred_element_type=jnp.float32)
    y = y + bias_ref[...]
    yo_ref[0] = y
    s, s2 = _stats_of(y)

    @pl.when(b == 0)
    def _():
        sto_ref[...] = jnp.zeros_like(sto_ref)

    sto_ref[0:1, :] += s
    sto_ref[1:2, :] += s2


def _run_mid_layer(y2d, st, g, be, wt, bias, c_out):
    c_in = y2d.shape[-1]
    return pl.pallas_call(
        _mid_layer_kernel,
        grid=(B,),
        in_specs=[
            pl.BlockSpec((1, NPOINT * K, c_in), lambda b: (b, 0, 0)),
            pl.BlockSpec((2, c_in), lambda b: (0, 0)),
            pl.BlockSpec((1, c_in), lambda b: (0, 0)),
            pl.BlockSpec((1, c_in), lambda b: (0, 0)),
            pl.BlockSpec((c_in, c_out), lambda b: (0, 0)),
            pl.BlockSpec((1, c_out), lambda b: (0, 0)),
        ],
        out_specs=[
            pl.BlockSpec((1, NPOINT * K, c_out), lambda b: (b, 0, 0)),
            pl.BlockSpec((2, c_out), lambda b: (0, 0)),
        ],
        out_shape=[
            jax.ShapeDtypeStruct((B, NPOINT * K, c_out), jnp.float32),
            jax.ShapeDtypeStruct((2, c_out), jnp.float32),
        ],
    )(y2d, st, g, be, wt, bias)


def _final_kernel(y_ref, st_ref, g_ref, be_ref, out_ref):
    scale, shift = _bn_scale_shift(st_ref, g_ref, be_ref)
    x = jnp.maximum(y_ref[0] * scale[None] + shift[None], 0.0)
    out_ref[0] = jnp.transpose(jnp.max(x, axis=1))


def _run_final(y3d, st, g, be):
    return pl.pallas_call(
        _final_kernel,
        grid=(B,),
        in_specs=[
            pl.BlockSpec((1, NPOINT, K, 64), lambda b: (b, 0, 0, 0)),
            pl.BlockSpec((2, 64), lambda b: (0, 0)),
            pl.BlockSpec((1, 64), lambda b: (0, 0)),
            pl.BlockSpec((1, 64), lambda b: (0, 0)),
        ],
        out_specs=pl.BlockSpec((1, 64, NPOINT), lambda b: (b, 0, 0)),
        out_shape=jax.ShapeDtypeStruct((B, 64, NPOINT), jnp.float32),
    )(y3d, st, g, be)


def kernel(xyz, points, w0, b0, g0, be0, w1, b1, g1, be1, w2, b2, g2, be2):
    xyz_c = jnp.transpose(xyz, (1, 0, 2))
    w0t = jnp.transpose(w0)
    w0x, w0p = w0t[:3], w0t[3:]

    new_xyz = jnp.transpose(_run_fps(xyz_c), (1, 0, 2))

    gidx, z_flat, qw = _run_knn(xyz, new_xyz.reshape(B * NPOINT, 3),
                                points, w0x, w0p)
    qw3 = qw.reshape(B, NPOINT, 32)

    gathered = _run_sc_gather(z_flat, gidx.reshape(-1))
    g4 = gathered.reshape(B, NPOINT, K, 128)

    y0, st0 = _run_layer0(g4, qw3, b0.reshape(1, 32))
    y1, st1 = _run_mid_layer(y0.reshape(B, NPOINT * K, 32), st0,
                             g0.reshape(1, 32), be0.reshape(1, 32),
                             jnp.transpose(w1), b1.reshape(1, 32), 32)
    y2, st2 = _run_mid_layer(y1, st1, g1.reshape(1, 32), be1.reshape(1, 32),
                             jnp.transpose(w2), b2.reshape(1, 64), 64)
    agg = _run_final(y2.reshape(B, NPOINT, K, 64), st2,
                     g2.reshape(1, 64), be2.reshape(1, 64))

    new_xyz_out = jnp.transpose(new_xyz, (0, 2, 1))
    return (new_xyz_out, agg)


# speedup vs baseline: 13.3301x; 1.0004x over previous
import functools

import jax
import jax.numpy as jnp
from jax import lax
from jax.experimental import pallas as pl
from jax.experimental.pallas import tpu as pltpu
from jax.experimental.pallas import tpu_sc as plsc

NPOINT = 512
K = 32
N = 4096
B = 8
C_IN = 32
EPS = 1e-5
BSK = B * NPOINT * K


def _fps_kernel(xyzc_ref, newxyz_ref):
    xc = xyzc_ref[...]
    x0 = xc[0]
    x1 = xc[1]
    x2 = xc[2]
    lane = lax.broadcasted_iota(jnp.int32, (B, N), 1)

    def fps_body(i, carry):
        dist, far = carry
        oh = lane == far
        cx = jnp.sum(jnp.where(oh, x0, 0.0), axis=1, keepdims=True)
        cy = jnp.sum(jnp.where(oh, x1, 0.0), axis=1, keepdims=True)
        cz = jnp.sum(jnp.where(oh, x2, 0.0), axis=1, keepdims=True)
        newxyz_ref[pl.ds(i, 1)] = jnp.concatenate([cx, cy, cz], axis=1)[None]
        d = (x0 - cx) ** 2 + (x1 - cy) ** 2 + (x2 - cz) ** 2
        dist = jnp.minimum(dist, d)
        m = jnp.max(dist, axis=1, keepdims=True)
        far2 = jnp.min(jnp.where(dist == m, lane, N), axis=1,
                       keepdims=True).astype(jnp.int32)
        return dist, far2

    dist0 = jnp.full((B, N), 1e10, jnp.float32)
    far0 = jnp.zeros((B, 1), jnp.int32)
    lax.fori_loop(0, NPOINT, fps_body, (dist0, far0))


def _run_fps(xyz_c):
    return pl.pallas_call(
        _fps_kernel,
        out_shape=jax.ShapeDtypeStruct((NPOINT, B, 3), jnp.float32),
    )(xyz_c)


def _knn_kernel(xyz_ref, nxyz_ref, pts_ref, w0x_ref, w0p_ref,
                gidx_ref, z_ref, qw_ref, dmat_ref):
    b = pl.program_id(0)

    q = nxyz_ref[...]
    x = xyz_ref[0]
    dots = lax.dot_general(q, x, (((1,), (0,)), ((), ())),
                           preferred_element_type=jnp.float32)
    qq = jnp.sum(q * q, axis=1, keepdims=True)
    xx = jnp.sum(x * x, axis=0, keepdims=True)
    dmat_ref[...] = (qq - 2.0 * dots) + xx

    TR = 128
    lane_full = lax.broadcasted_iota(jnp.int32, (TR, N), 1)
    col32 = lax.broadcasted_iota(jnp.int32, (TR, K), 1)
    inf = jnp.float32(3.0e38)
    NT = NPOINT // TR

    def knn_body(j, accs):
        new_accs = []
        for t in range(NT):
            rows = pl.ds(t * TR, TR)
            dt = dmat_ref[rows, :]
            m = jnp.min(dt, axis=1, keepdims=True)
            idx = jnp.min(jnp.where(dt == m, lane_full, N), axis=1,
                          keepdims=True).astype(jnp.int32)
            dmat_ref[rows, :] = jnp.where(lane_full == idx, inf, dt)
            new_accs.append(jnp.where(col32 == j, idx, accs[t]))
        return tuple(new_accs)

    acc0 = tuple(jnp.zeros((TR, K), jnp.int32) for _ in range(NT))
    accs = lax.fori_loop(0, K, knn_body, acc0)
    for t in range(NT):
        gidx_ref[pl.ds(t * TR, TR), :] = accs[t] + b * N

    w0x = w0x_ref[...]
    z = (lax.dot_general(x, w0x, (((0,), (0,)), ((), ())),
                         preferred_element_type=jnp.float32)
         + lax.dot_general(pts_ref[0], w0p_ref[...], (((0,), (0,)), ((), ())),
                           preferred_element_type=jnp.float32))
    z_ref[...] = jnp.concatenate(
        [z, jnp.zeros((N, 96), jnp.float32)], axis=1)
    qw_ref[...] = lax.dot_general(q, w0x, (((1,), (0,)), ((), ())),
                                  preferred_element_type=jnp.float32)


def _run_knn(xyz, new_xyz_flat, points, w0x, w0p):
    return pl.pallas_call(
        _knn_kernel,
        grid=(B,),
        in_specs=[
            pl.BlockSpec((1, 3, N), lambda b: (b, 0, 0)),
            pl.BlockSpec((NPOINT, 3), lambda b: (b, 0)),
            pl.BlockSpec((1, C_IN, N), lambda b: (b, 0, 0)),
            pl.BlockSpec((3, 32), lambda b: (0, 0)),
            pl.BlockSpec((C_IN, 32), lambda b: (0, 0)),
        ],
        out_specs=[
            pl.BlockSpec((NPOINT, K), lambda b: (b, 0)),
            pl.BlockSpec((N, 128), lambda b: (b, 0)),
            pl.BlockSpec((NPOINT, 32), lambda b: (b, 0)),
        ],
        out_shape=[
            jax.ShapeDtypeStruct((B * NPOINT, K), jnp.int32),
            jax.ShapeDtypeStruct((B * N, 128), jnp.float32),
            jax.ShapeDtypeStruct((B * NPOINT, 32), jnp.float32),
        ],
        scratch_shapes=[pltpu.VMEM((NPOINT, N), jnp.float32)],
    )(xyz, new_xyz_flat, points, w0x, w0p)


_SC_NW = 32
_SC_BPW = (B * NPOINT * K) // _SC_NW
_SC_CHUNK = 512


def _sc_gather_kernel(z_hbm, idx_hbm, out_hbm, idx_v, rows_v, sem):
    wid = lax.axis_index("s") * 2 + lax.axis_index("c")
    base = wid * _SC_BPW
    pltpu.sync_copy(idx_hbm.at[pl.ds(base, _SC_BPW)], idx_v)
    for c in range(_SC_BPW // _SC_CHUNK):
        pltpu.async_copy(
            z_hbm.at[idx_v.at[pl.ds(c * _SC_CHUNK, _SC_CHUNK)]],
            rows_v, sem).wait()
        pltpu.sync_copy(rows_v,
                        out_hbm.at[pl.ds(base + c * _SC_CHUNK, _SC_CHUNK)])


def _run_sc_gather(z_flat, idx_flat):
    mesh = plsc.VectorSubcoreMesh(core_axis_name="c", subcore_axis_name="s")
    k = functools.partial(
        pl.kernel, mesh=mesh,
        out_type=jax.ShapeDtypeStruct((B * NPOINT * K, 128), jnp.float32),
        scratch_types=[
            pltpu.VMEM((_SC_BPW,), jnp.int32),
            pltpu.VMEM((_SC_CHUNK, 128), jnp.float32),
            pltpu.SemaphoreType.DMA,
        ],
    )(_sc_gather_kernel)
    return k(z_flat, idx_flat)


def _stats_of(y):
    if y.ndim == 3:
        y = jnp.reshape(y, (-1, y.shape[-1]))
    rows, c = y.shape
    ones = jnp.ones((1, rows), jnp.float32)
    s = lax.dot_general(ones, y, (((1,), (0,)), ((), ())),
                        preferred_element_type=jnp.float32)
    yy = lax.dot_general(y, y, (((0,), (0,)), ((), ())),
                         preferred_element_type=jnp.float32)
    eye = jnp.eye(c, dtype=jnp.float32)
    s2 = jnp.sum(yy * eye, axis=0, keepdims=True)
    return s, s2


def _layer0_kernel(g_ref, qw_ref, b0_ref, y_ref, st_ref):
    b = pl.program_id(0)
    g = g_ref[0][:, :, 0:32]
    qw = qw_ref[0]
    y = g - qw[:, None, :] + b0_ref[...][None, :, :]
    y_ref[0] = y
    s, s2 = _stats_of(y)

    @pl.when(b == 0)
    def _():
        st_ref[...] = jnp.zeros_like(st_ref)

    st_ref[0:1, :] += s
    st_ref[1:2, :] += s2


def _run_layer0(g4, qw3, b0):
    return pl.pallas_call(
        _layer0_kernel,
        grid=(B,),
        in_specs=[
            pl.BlockSpec((1, NPOINT, K, 128), lambda b: (b, 0, 0, 0)),
            pl.BlockSpec((1, NPOINT, 32), lambda b: (b, 0, 0)),
            pl.BlockSpec((1, 32), lambda b: (0, 0)),
        ],
        out_specs=[
            pl.BlockSpec((1, NPOINT, K, 32), lambda b: (b, 0, 0, 0)),
            pl.BlockSpec((2, 32), lambda b: (0, 0)),
        ],
        out_shape=[
            jax.ShapeDtypeStruct((B, NPOINT, K, 32), jnp.float32),
            jax.ShapeDtypeStruct((2, 32), jnp.float32),
        ],
    )(g4, qw3, b0)


def _bn_scale_shift(st_ref, g_ref, be_ref):
    mean = st_ref[0:1, :] / BSK
    var = st_ref[1:2, :] / BSK - mean * mean
    scale = g_ref[...] / jnp.sqrt(var + EPS)
    shift = be_ref[...] - mean * scale
    return scale, shift


def _mid_layer_kernel(y_ref, st_ref, g_ref, be_ref, wt_ref, bias_ref,
                      yo_ref, sto_ref):
    b = pl.program_id(0)
    scale, shift = _bn_scale_shift(st_ref, g_ref, be_ref)
    x = jnp.maximum(y_ref[0] * scale + shift, 0.0)
    y = lax.dot_general(x, wt_ref[...], (((1,), (0,)), ((), ())),
                        preferred_element_type=jnp.float32)
    y = y + bias_ref[...]
    yo_ref[0] = y
    s, s2 = _stats_of(y)

    @pl.when(b == 0)
    def _():
        sto_ref[...] = jnp.zeros_like(sto_ref)

    sto_ref[0:1, :] += s
    sto_ref[1:2, :] += s2


def _run_mid_layer(y2d, st, g, be, wt, bias, c_out):
    c_in = y2d.shape[-1]
    return pl.pallas_call(
        _mid_layer_kernel,
        grid=(B,),
        in_specs=[
            pl.BlockSpec((1, NPOINT * K, c_in), lambda b: (b, 0, 0)),
            pl.BlockSpec((2, c_in), lambda b: (0, 0)),
            pl.BlockSpec((1, c_in), lambda b: (0, 0)),
            pl.BlockSpec((1, c_in), lambda b: (0, 0)),
            pl.BlockSpec((c_in, c_out), lambda b: (0, 0)),
            pl.BlockSpec((1, c_out), lambda b: (0, 0)),
        ],
        out_specs=[
            pl.BlockSpec((1, NPOINT * K, c_out), lambda b: (b, 0, 0)),
            pl.BlockSpec((2, c_out), lambda b: (0, 0)),
        ],
        out_shape=[
            jax.ShapeDtypeStruct((B, NPOINT * K, c_out), jnp.float32),
            jax.ShapeDtypeStruct((2, c_out), jnp.float32),
        ],
    )(y2d, st, g, be, wt, bias)


def _final_kernel(y_ref, st_ref, g_ref, be_ref, out_ref):
    scale, shift = _bn_scale_shift(st_ref, g_ref, be_ref)
    x = jnp.maximum(y_ref[0] * scale[None] + shift[None], 0.0)
    out_ref[0] = jnp.transpose(jnp.max(x, axis=1))


def _run_final(y3d, st, g, be):
    return pl.pallas_call(
        _final_kernel,
        grid=(B,),
        in_specs=[
            pl.BlockSpec((1, NPOINT, K, 64), lambda b: (b, 0, 0, 0)),
            pl.BlockSpec((2, 64), lambda b: (0, 0)),
            pl.BlockSpec((1, 64), lambda b: (0, 0)),
            pl.BlockSpec((1, 64), lambda b: (0, 0)),
        ],
        out_specs=pl.BlockSpec((1, 64, NPOINT), lambda b: (b, 0, 0)),
        out_shape=jax.ShapeDtypeStruct((B, 64, NPOINT), jnp.float32),
    )(y3d, st, g, be)


def kernel(xyz, points, w0, b0, g0, be0, w1, b1, g1, be1, w2, b2, g2, be2):
    xyz_c = jnp.transpose(xyz, (1, 0, 2))
    w0t = jnp.transpose(w0)
    w0x, w0p = w0t[:3], w0t[3:]

    new_xyz = jnp.transpose(_run_fps(xyz_c), (1, 0, 2))

    gidx, z_flat, qw = _run_knn(xyz, new_xyz.reshape(B * NPOINT, 3),
                                points, w0x, w0p)
    qw3 = qw.reshape(B, NPOINT, 32)

    gathered = _run_sc_gather(z_flat, gidx.reshape(-1))
    g4 = gathered.reshape(B, NPOINT, K, 128)

    y0, st0 = _run_layer0(g4, qw3, b0.reshape(1, 32))
    y1, st1 = _run_mid_layer(y0.reshape(B, NPOINT * K, 32), st0,
                             g0.reshape(1, 32), be0.reshape(1, 32),
                             jnp.transpose(w1), b1.reshape(1, 32), 32)
    y2, st2 = _run_mid_layer(y1, st1, g1.reshape(1, 32), be1.reshape(1, 32),
                             jnp.transpose(w2), b2.reshape(1, 64), 64)
    agg = _run_final(y2.reshape(B, NPOINT, K, 64), st2,
                     g2.reshape(1, 64), be2.reshape(1, 64))

    new_xyz_out = jnp.transpose(new_xyz, (0, 2, 1))
    return (new_xyz_out, agg)
